# Initial kernel scaffold; baseline (speedup 1.0000x reference)
import jax
import jax.numpy as jnp
from jax.experimental import pallas as pl


def kernel(x_user, x_item, edge_index_user_item, edge_index_item_user, Wl_u2i, Wr_u2i, b_u2i, Wl_i2u, Wr_i2u, b_i2u):
    raise NotImplementedError("write your pallas kernel here")


# speedup vs baseline: 4.0950x; 4.0950x over previous
import functools

import jax
import jax.numpy as jnp
from jax import lax
from jax.experimental import pallas as pl
from jax.experimental.pallas import tpu as pltpu, tpu_sc as plsc

N = 50000
D = 128
E = 400000

NC, NS = 2, 16
SL = 25600
E_PAD = SL * NS
CH = 2560
NBLK = 8
BN = 6304
N2 = NBLK * BN
BNP = 6312
GB = 128
NR = 32
ZCH = 8


def _sc_segment_sum(x_src, src_pad, dst_pad):
  mesh = plsc.VectorSubcoreMesh(
      core_axis_name="c", subcore_axis_name="s",
      num_cores=NC, num_subcores=NS)

  @functools.partial(
      pl.kernel,
      out_type=(
          jax.ShapeDtypeStruct((N2, D), jnp.float32),
          jax.ShapeDtypeStruct((N2, 16), jnp.float32),
      ),
      mesh=mesh,
      compiler_params=pltpu.CompilerParams(
          needs_layout_passes=False, use_tc_tiling_on_sc=False),
      scratch_types=[
          pltpu.VMEM((CH,), jnp.int32),
          pltpu.VMEM((CH,), jnp.int32),
          pltpu.VMEM((NR, GB), jnp.int32),
          pltpu.VMEM((NR, GB), jnp.int32),
          pltpu.VMEM((GB, D), jnp.float32),
          pltpu.VMEM((GB, 16), jnp.float32),
          pltpu.VMEM((ZCH, D), jnp.float32),
          pltpu.VMEM((ZCH, 16), jnp.float32),
          pltpu.VMEM_SHARED((BNP, D), jnp.float32),
          pltpu.VMEM_SHARED((BNP, 16), jnp.float32),
      ],
  )
  def seg_kernel(xsrc_hbm, src_hbm, dst_hbm, out_sum, out_cnt,
                 src_c, dst_c, keep_src, keep_dst, rows_v, ones_v,
                 zrows, zcnt, acc_sh, cnt_sh):
    c = lax.axis_index("c")
    s = lax.axis_index("s")
    zero16f = jnp.zeros((16,), jnp.float32)
    one16f = jnp.ones((16,), jnp.float32)
    iota16 = lax.iota(jnp.int32, 16)

    def init_z(r, _):
      for cc in range(D // 16):
        zrows[r, pl.ds(cc * 16, 16)] = zero16f
      zcnt[r, pl.ds(0, 16)] = zero16f
      return 0
    lax.fori_loop(0, ZCH, init_z, 0)
    def init_ones(r, _):
      ones_v[r, pl.ds(0, 16)] = one16f
      return 0
    lax.fori_loop(0, GB, init_ones, 0)

    base = s * SL
    for blk in range(NBLK // NC):
      lo = (c * (NBLK // NC) + blk) * BN

      r0 = s * 400
      def zero_acc(i, _):
        pltpu.sync_copy(zrows, acc_sh.at[pl.ds(r0 + i * ZCH, ZCH)])
        pltpu.sync_copy(zcnt, cnt_sh.at[pl.ds(r0 + i * ZCH, ZCH)])
        return 0
      nchunk = jnp.where(s < NS - 1, 400 // ZCH, 312 // ZCH)
      lax.fori_loop(0, nchunk, zero_acc, 0)
      plsc.subcore_barrier()

      def chunk_body(k, carry):
        w, dr = carry
        cix = s * (SL // CH) + k
        pltpu.sync_copy(src_hbm.at[cix], src_c)
        pltpu.sync_copy(dst_hbm.at[cix], dst_c)

        def compact(g, w):
          dv = dst_c[pl.ds(g * 16, 16)]
          sv = src_c[pl.ds(g * 16, 16)]
          dl = dv - lo
          m = (dl >= 0) & (dl < BN)
          mi = m.astype(jnp.int32)
          pos = w + plsc.cumsum(mi) - 1
          pr = (pos >> 7) & (NR - 1)
          pc = pos & (GB - 1)
          plsc.store_scatter(keep_src, [pr, pc], sv, mask=m)
          plsc.store_scatter(keep_dst, [pr, pc], dl, mask=m)
          return w + jnp.sum(mi)
        w = lax.fori_loop(0, CH // 16, compact, w)

        full = w >> 7
        def drain(b, _):
          br = b & (NR - 1)
          pltpu.sync_copy(xsrc_hbm.at[keep_src.at[br]], rows_v)
          pltpu.sync_copy(rows_v, acc_sh.at[keep_dst.at[br]], add=True)
          pltpu.sync_copy(ones_v, cnt_sh.at[keep_dst.at[br]], add=True)
          return 0
        lax.fori_loop(dr, full, drain, 0)
        return (w, full)

      w, dr = lax.fori_loop(0, SL // CH, chunk_body,
                            (jnp.int32(0), jnp.int32(0)))

      trash = BN + (iota16 & (BNP - BN - 1))
      for i in range(GB // 16):
        p = w + i * 16 + iota16
        pr = (p >> 7) & (NR - 1)
        pc = p & (GB - 1)
        plsc.store_scatter(keep_src, [pr, pc], iota16)
        plsc.store_scatter(keep_dst, [pr, pc], trash)
      br = dr & (NR - 1)
      pltpu.sync_copy(xsrc_hbm.at[keep_src.at[br]], rows_v)
      pltpu.sync_copy(rows_v, acc_sh.at[keep_dst.at[br]], add=True)
      pltpu.sync_copy(ones_v, cnt_sh.at[keep_dst.at[br]], add=True)
      plsc.subcore_barrier()

      @pl.when(s < NS - 1)
      def _():
        o0 = s * 400
        pltpu.sync_copy(acc_sh.at[pl.ds(o0, 400)], out_sum.at[pl.ds(lo + o0, 400)])
        pltpu.sync_copy(cnt_sh.at[pl.ds(o0, 400)], out_cnt.at[pl.ds(lo + o0, 400)])
      @pl.when(s == NS - 1)
      def _():
        o0 = 15 * 400
        pltpu.sync_copy(acc_sh.at[pl.ds(o0, 304)], out_sum.at[pl.ds(lo + o0, 304)])
        pltpu.sync_copy(cnt_sh.at[pl.ds(o0, 304)], out_cnt.at[pl.ds(lo + o0, 304)])
      plsc.subcore_barrier()


  return seg_kernel(x_src, src_pad, dst_pad)


def _tc_sage_out(summed, cnt16, x_dst, Wl, Wr, b):
  BLK = 400

  def body(sum_ref, cnt_ref, x_ref, wl_ref, wr_ref, b_ref, out_ref):
    cnt = jnp.maximum(cnt_ref[:, 0:1], 1.0)
    mean = sum_ref[...] / cnt
    out_ref[...] = (
        jnp.dot(mean, wl_ref[...], preferred_element_type=jnp.float32)
        + jnp.dot(x_ref[...], wr_ref[...], preferred_element_type=jnp.float32)
        + b_ref[...])

  return pl.pallas_call(
      body,
      grid=(N // BLK,),
      in_specs=[
          pl.BlockSpec((BLK, D), lambda i: (i, 0)),
          pl.BlockSpec((BLK, 16), lambda i: (i, 0)),
          pl.BlockSpec((BLK, D), lambda i: (i, 0)),
          pl.BlockSpec((D, D), lambda i: (0, 0)),
          pl.BlockSpec((D, D), lambda i: (0, 0)),
          pl.BlockSpec((1, D), lambda i: (0, 0)),
      ],
      out_specs=pl.BlockSpec((BLK, D), lambda i: (i, 0)),
      out_shape=jax.ShapeDtypeStruct((N, D), jnp.float32),
  )(summed, cnt16, x_dst, Wl, Wr, b.reshape(1, D))


def _pad_edges(edge_index):
  src = edge_index[0].astype(jnp.int32)
  dst = edge_index[1].astype(jnp.int32)
  pad = E_PAD - E
  src = jnp.concatenate([src, jnp.zeros((pad,), jnp.int32)])
  dst = jnp.concatenate([dst, jnp.full((pad,), 1 << 29, jnp.int32)])
  return src.reshape(E_PAD // CH, CH), dst.reshape(E_PAD // CH, CH)


def kernel(x_user, x_item, edge_index_user_item, edge_index_item_user,
           Wl_u2i, Wr_u2i, b_u2i, Wl_i2u, Wr_i2u, b_i2u):
  src_ui, dst_ui = _pad_edges(edge_index_user_item)
  src_iu, dst_iu = _pad_edges(edge_index_item_user)

  sum_item, cnt_item = _sc_segment_sum(x_user, src_ui, dst_ui)
  sum_user, cnt_user = _sc_segment_sum(x_item, src_iu, dst_iu)

  out_item = _tc_sage_out(sum_item, cnt_item, x_item, Wl_u2i, Wr_u2i, b_u2i)
  out_user = _tc_sage_out(sum_user, cnt_user, x_user, Wl_i2u, Wr_i2u, b_i2u)
  return (out_user, out_item)
